# baseline (speedup 1.0000x reference)
import functools

import jax
import jax.numpy as jnp
from jax import lax
from jax.experimental import pallas as pl
from jax.experimental.pallas import tpu as pltpu
from jax.experimental.pallas import tpu_sc as plsc

_INV_T = 10.0
_L = 16
_PAD_VAL = -6.25e28


@functools.lru_cache(maxsize=None)
def _make_fmt(V, D, NC, NS):
    NW = NC * NS
    assert D == 4 * _L
    BK = 256
    NBF = V // BK
    TAIL = V % BK
    per = (NBF + NW - 1) // NW
    mesh = plsc.VectorSubcoreMesh(
        core_axis_name="c", subcore_axis_name="s",
        num_cores=NC, num_subcores=NS)

    def body(t_t, u_t, t_tail, u_tail, item_lin, user_lin,
             bin0, bin1, bout0, bout1, semi0, semi1, semo0, semo1):
        w = lax.axis_index("s") * NC + lax.axis_index("c")
        lane = lax.iota(jnp.int32, _L)
        rots = tuple(jnp.bitwise_and(lane + s, _L - 1) for s in range(_L))

        bins = (bin0, bin1)
        bouts = (bout0, bout1)
        semis = (semi0, semi1)
        semos = (semo0, semo1)

        dls = tuple(d0 + lane for d0 in range(0, D, _L))

        def transpose_block(src, dst):
            def tbody(rg, _):
                r0 = rg * _L
                for s in range(_L):
                    cv = r0 + rots[s]
                    cvd = cv * D
                    for k in range(D // _L):
                        v = plsc.load_gather(src, [dls[k], cv])
                        plsc.store_scatter(dst, [cvd + dls[k]], v)
                return _
            lax.fori_loop(0, BK // _L, tbody, 0, unroll=4)

        for tbl, tail, out in ((t_t, t_tail, item_lin),
                               (u_t, u_tail, user_lin)):
            c0 = w * per
            n = jnp.minimum(per, NBF - c0)

            def fetch(c, slot):
                pltpu.async_copy(
                    tbl.at[:, pl.ds(c * BK, BK)], bins[slot], semis[slot])

            def wait_in(slot):
                pltpu.make_async_copy(
                    tbl.at[:, pl.ds(0, BK)], bins[slot], semis[slot]).wait()

            def wait_out(slot):
                pltpu.make_async_copy(
                    bouts[slot], out.at[pl.ds(0, BK * D)], semos[slot]).wait()

            @pl.when(n > 0)
            def _prime():
                fetch(c0, 0)

            def outer(i, _):
                for k in range(2):
                    li = i * 2 + k
                    c = c0 + li

                    @pl.when(li + 1 < n)
                    def _pf():
                        fetch(c + 1, 1 - k)

                    @pl.when(li < n)
                    def _do():
                        wait_in(k)

                        @pl.when(li >= 2)
                        def _wo():
                            wait_out(k)

                        transpose_block(bins[k], bouts[k])
                        pltpu.async_copy(
                            bouts[k], out.at[pl.ds(c * BK * D, BK * D)],
                            semos[k])
                return _

            lax.fori_loop(0, (per + 1) // 2, outer, 0, unroll=False)
            for k in range(2):
                @pl.when((n + 1 - k) // 2 > 0)
                def _drain():
                    wait_out(k)

            if TAIL:
                @pl.when(w == NW - 1)
                def _tail():
                    pltpu.sync_copy(tail, bouts[1].at[pl.ds(0, TAIL * D)])
                    pltpu.sync_copy(
                        bouts[1].at[pl.ds(0, TAIL * D)],
                        out.at[pl.ds(NBF * BK * D, TAIL * D)])

    return pl.kernel(
        body,
        out_type=(jax.ShapeDtypeStruct((V * D,), jnp.float32),
                  jax.ShapeDtypeStruct((V * D,), jnp.float32)),
        mesh=mesh,
        compiler_params=pltpu.CompilerParams(
            needs_layout_passes=False, use_tc_tiling_on_sc=True),
        scratch_types=[
            pltpu.VMEM((D, 256), jnp.float32),
            pltpu.VMEM((D, 256), jnp.float32),
            pltpu.VMEM((256 * D,), jnp.float32),
            pltpu.VMEM((256 * D,), jnp.float32),
            pltpu.SemaphoreType.DMA,
            pltpu.SemaphoreType.DMA,
            pltpu.SemaphoreType.DMA,
            pltpu.SemaphoreType.DMA,
        ],
    )


@functools.lru_cache(maxsize=None)
def _make_sc_scores(B, N, D, NC, NS):
    NW = NC * NS
    assert B % NW == 0 and D == 4 * _L
    bw = B // NW
    assert bw % _L == 0
    n_full = N // _L
    n_tail = N % _L
    mesh = plsc.VectorSubcoreMesh(
        core_axis_name="c", subcore_axis_name="s",
        num_cores=NC, num_subcores=NS)

    def body(users_hbm, pos_hbm, neg_hbm, uemb_hbm, iemb_hbm, out_hbm,
             u_idx, p_idx, u_rows, p_rows,
             n_idx0, n_idx1, n_rows0, n_rows1,
             tbuf, tbuf2, sbuf, pbuf, res, sem0, sem1):
        wid = lax.axis_index("s") * NC + lax.axis_index("c")
        base = wid * bw
        lane = lax.iota(jnp.int32, _L)
        lane_l = lane * _L

        def treduce(buf):
            acc = plsc.load_gather(buf, [lane_l])
            for c in range(1, _L):
                acc = acc + plsc.load_gather(buf, [lane_l + c])
            return acc

        pltpu.sync_copy(users_hbm.at[pl.ds(base, bw)], u_idx)
        pltpu.sync_copy(pos_hbm.at[pl.ds(base, bw)], p_idx)
        pltpu.async_copy(uemb_hbm.at[u_idx], u_rows, sem0).wait()
        pltpu.async_copy(iemb_hbm.at[p_idx], p_rows, sem0).wait()

        if n_tail:
            for r in range(n_tail, _L):
                tbuf2[pl.ds(r * _L, _L)] = jnp.full((_L,), _PAD_VAL,
                                                    jnp.float32)

        n_idx = (n_idx0, n_idx1)
        n_rows = (n_rows0, n_rows1)
        sems = (sem0, sem1)

        def fetch(b, slot):
            pltpu.sync_copy(neg_hbm.at[base + b], n_idx[slot])
            pltpu.async_copy(iemb_hbm.at[n_idx[slot]], n_rows[slot],
                             sems[slot])

        def compute(b, slot):
            rows = n_rows[slot]
            u0 = u_rows[b, pl.ds(0, _L)]
            u1 = u_rows[b, pl.ds(_L, _L)]
            u2 = u_rows[b, pl.ds(2 * _L, _L)]
            u3 = u_rows[b, pl.ds(3 * _L, _L)]

            def dot_to(buf, r, j):
                t = (u0 * rows[j, pl.ds(0, _L)]
                     + u1 * rows[j, pl.ds(_L, _L)]
                     + u2 * rows[j, pl.ds(2 * _L, _L)]
                     + u3 * rows[j, pl.ds(3 * _L, _L)])
                buf[pl.ds(r * _L, _L)] = t

            def gbody(g, sacc):
                for r in range(_L):
                    dot_to(tbuf, r, g * _L + r)
                return sacc + jnp.exp(treduce(tbuf) * _INV_T)

            sacc = lax.fori_loop(0, n_full, gbody,
                                 jnp.zeros((_L,), jnp.float32), unroll=4)
            if n_tail:
                for r in range(n_tail):
                    dot_to(tbuf2, r, n_full * _L + r)
                sacc = sacc + jnp.exp(treduce(tbuf2) * _INV_T)

            tp = (u0 * p_rows[b, pl.ds(0, _L)]
                  + u1 * p_rows[b, pl.ds(_L, _L)]
                  + u2 * p_rows[b, pl.ds(2 * _L, _L)]
                  + u3 * p_rows[b, pl.ds(3 * _L, _L)])
            bm = jnp.bitwise_and(b, _L - 1)
            sbuf[pl.ds(bm * _L, _L)] = sacc
            pbuf[pl.ds(bm * _L, _L)] = tp

            @pl.when(bm == _L - 1)
            def _fold():
                svec = treduce(sbuf)
                pe = jnp.exp(treduce(pbuf) * _INV_T)
                res[pl.ds(b - (_L - 1), _L)] = pe / (pe + svec)

        fetch(0, 0)

        def outer(i, _):
            for k in range(2):
                b = i * 2 + k
                slot = k

                @pl.when(b + 1 < bw)
                def _prefetch():
                    fetch(b + 1, 1 - slot)

                pltpu.make_async_copy(
                    iemb_hbm.at[n_idx[slot]], n_rows[slot],
                    sems[slot]).wait()
                compute(b, slot)
            return _

        lax.fori_loop(0, bw // 2, outer, 0, unroll=False)
        pltpu.sync_copy(res, out_hbm.at[pl.ds(base, bw)])

    return pl.kernel(
        body,
        out_type=jax.ShapeDtypeStruct((B,), jnp.float32),
        mesh=mesh,
        compiler_params=pltpu.CompilerParams(
            needs_layout_passes=False, use_tc_tiling_on_sc=False),
        scratch_types=[
            pltpu.VMEM((bw,), jnp.int32),
            pltpu.VMEM((bw,), jnp.int32),
            pltpu.VMEM((bw, D), jnp.float32),
            pltpu.VMEM((bw, D), jnp.float32),
            pltpu.VMEM((N,), jnp.int32),
            pltpu.VMEM((N,), jnp.int32),
            pltpu.VMEM((N, D), jnp.float32),
            pltpu.VMEM((N, D), jnp.float32),
            pltpu.VMEM((_L * _L,), jnp.float32),
            pltpu.VMEM((_L * _L,), jnp.float32),
            pltpu.VMEM((_L * _L,), jnp.float32),
            pltpu.VMEM((_L * _L,), jnp.float32),
            pltpu.VMEM((bw,), jnp.float32),
            pltpu.SemaphoreType.DMA,
            pltpu.SemaphoreType.DMA,
        ],
    )


def _loss_body(r_ref, o_ref):
    o_ref[0, 0] = -jnp.mean(jnp.log(r_ref[...]))


@functools.lru_cache(maxsize=None)
def _make_tc_loss(rows, cols):
    return pl.pallas_call(
        _loss_body,
        out_shape=jax.ShapeDtypeStruct((1, 1), jnp.float32),
        out_specs=pl.BlockSpec(memory_space=pltpu.SMEM),
    )


def kernel(users, positives, negatives, user_emb, item_emb, epoch):
    B = users.shape[0]
    N = negatives.shape[1]
    V, D = item_emb.shape
    info = plsc.get_sparse_core_info()
    NC, NS = info.num_cores, info.num_subcores
    tail = V % 256
    item_lin, user_lin = _make_fmt(V, D, NC, NS)(
        jnp.transpose(item_emb), jnp.transpose(user_emb),
        item_emb[V - tail:, :].reshape(-1), user_emb[V - tail:, :].reshape(-1))
    sc = _make_sc_scores(B, N, D, NC, NS)
    r = sc(users.astype(jnp.int32),
           positives.reshape(-1).astype(jnp.int32),
           negatives.astype(jnp.int32),
           user_lin.reshape(V, D), item_lin.reshape(V, D))
    loss = _make_tc_loss(B // 128, 128)(r.reshape(B // 128, 128))
    return loss[0, 0]


# speedup vs baseline: 1.0462x; 1.0462x over previous
import functools

import jax
import jax.numpy as jnp
from jax import lax
from jax.experimental import pallas as pl
from jax.experimental.pallas import tpu as pltpu
from jax.experimental.pallas import tpu_sc as plsc

_INV_T = 10.0
_L = 16
_PAD_VAL = -6.25e28


@functools.lru_cache(maxsize=None)
def _make_fmt(V, D, NC, NS):
    NW = NC * NS
    assert D == 4 * _L
    BK = 256
    NBF = V // BK
    TAIL = V % BK
    per = (NBF + NW - 1) // NW
    mesh = plsc.VectorSubcoreMesh(
        core_axis_name="c", subcore_axis_name="s",
        num_cores=NC, num_subcores=NS)

    def body(t_t, u_t, t_tail, u_tail, item_lin, user_lin,
             bin0, bin1, bout0, bout1, semi0, semi1, semo0, semo1):
        w = lax.axis_index("s") * NC + lax.axis_index("c")
        lane = lax.iota(jnp.int32, _L)
        rots = tuple(jnp.bitwise_and(lane + s, _L - 1) for s in range(_L))

        bins = (bin0, bin1)
        bouts = (bout0, bout1)
        semis = (semi0, semi1)
        semos = (semo0, semo1)

        dls = tuple(d0 + lane for d0 in range(0, D, _L))

        def transpose_block(src, dst):
            def tbody(rg, _):
                r0 = rg * _L
                for s in range(_L):
                    cv = r0 + rots[s]
                    cvd = cv * D
                    for k in range(D // _L):
                        v = plsc.load_gather(src, [dls[k], cv])
                        plsc.store_scatter(dst, [cvd + dls[k]], v)
                return _
            lax.fori_loop(0, BK // _L, tbody, 0, unroll=2)

        for tbl, tail, out in ((t_t, t_tail, item_lin),
                               (u_t, u_tail, user_lin)):
            c0 = w * per
            n = jnp.minimum(per, NBF - c0)

            def fetch(c, slot):
                pltpu.async_copy(
                    tbl.at[:, pl.ds(c * BK, BK)], bins[slot], semis[slot])

            def wait_in(slot):
                pltpu.make_async_copy(
                    tbl.at[:, pl.ds(0, BK)], bins[slot], semis[slot]).wait()

            def wait_out(slot):
                pltpu.make_async_copy(
                    bouts[slot], out.at[pl.ds(0, BK * D)], semos[slot]).wait()

            @pl.when(n > 0)
            def _prime():
                fetch(c0, 0)

            def outer(i, _):
                for k in range(2):
                    li = i * 2 + k
                    c = c0 + li

                    @pl.when(li + 1 < n)
                    def _pf():
                        fetch(c + 1, 1 - k)

                    @pl.when(li < n)
                    def _do():
                        wait_in(k)

                        @pl.when(li >= 2)
                        def _wo():
                            wait_out(k)

                        transpose_block(bins[k], bouts[k])
                        pltpu.async_copy(
                            bouts[k], out.at[pl.ds(c * BK * D, BK * D)],
                            semos[k])
                return _

            lax.fori_loop(0, (per + 1) // 2, outer, 0, unroll=False)
            for k in range(2):
                @pl.when((n + 1 - k) // 2 > 0)
                def _drain():
                    wait_out(k)

            if TAIL:
                @pl.when(w == NW - 1)
                def _tail():
                    pltpu.sync_copy(tail, bouts[1].at[pl.ds(0, TAIL * D)])
                    pltpu.sync_copy(
                        bouts[1].at[pl.ds(0, TAIL * D)],
                        out.at[pl.ds(NBF * BK * D, TAIL * D)])

    return pl.kernel(
        body,
        out_type=(jax.ShapeDtypeStruct((V * D,), jnp.float32),
                  jax.ShapeDtypeStruct((V * D,), jnp.float32)),
        mesh=mesh,
        compiler_params=pltpu.CompilerParams(
            needs_layout_passes=False, use_tc_tiling_on_sc=True),
        scratch_types=[
            pltpu.VMEM((D, 256), jnp.float32),
            pltpu.VMEM((D, 256), jnp.float32),
            pltpu.VMEM((256 * D,), jnp.float32),
            pltpu.VMEM((256 * D,), jnp.float32),
            pltpu.SemaphoreType.DMA,
            pltpu.SemaphoreType.DMA,
            pltpu.SemaphoreType.DMA,
            pltpu.SemaphoreType.DMA,
        ],
    )


@functools.lru_cache(maxsize=None)
def _make_sc_scores(B, N, D, NC, NS):
    NW = NC * NS
    assert B % NW == 0 and D == 4 * _L
    bw = B // NW
    assert bw % _L == 0
    n_full = N // _L
    n_tail = N % _L
    mesh = plsc.VectorSubcoreMesh(
        core_axis_name="c", subcore_axis_name="s",
        num_cores=NC, num_subcores=NS)

    def body(users_hbm, pos_hbm, neg_hbm, uemb_hbm, iemb_hbm, out_hbm,
             u_idx, p_idx, u_rows, p_rows,
             n_idx0, n_idx1, n_rows0, n_rows1,
             tbuf, tbuf2, sbuf, pbuf, res, sem0, sem1):
        wid = lax.axis_index("s") * NC + lax.axis_index("c")
        base = wid * bw
        lane = lax.iota(jnp.int32, _L)
        lane_l = lane * _L
        rots = tuple(jnp.bitwise_and(lane + s, _L - 1) for s in range(_L))

        def rot_store(buf, r, t):
            buf[pl.ds(r * _L, _L)] = t.at[rots[r % _L]].get(mode="promise_in_bounds")

        def treduce(buf):
            acc = plsc.load_gather(buf, [lane_l + rots[0]])
            for c in range(1, _L):
                acc = acc + plsc.load_gather(buf, [lane_l + rots[c]])
            return acc

        pltpu.sync_copy(users_hbm.at[pl.ds(base, bw)], u_idx)
        pltpu.sync_copy(pos_hbm.at[pl.ds(base, bw)], p_idx)
        pltpu.async_copy(uemb_hbm.at[u_idx], u_rows, sem0).wait()
        pltpu.async_copy(iemb_hbm.at[p_idx], p_rows, sem0).wait()

        if n_tail:
            for r in range(n_tail, _L):
                tbuf2[pl.ds(r * _L, _L)] = jnp.full((_L,), _PAD_VAL,
                                                    jnp.float32)

        n_idx = (n_idx0, n_idx1)
        n_rows = (n_rows0, n_rows1)
        sems = (sem0, sem1)

        def fetch(b, slot):
            pltpu.sync_copy(neg_hbm.at[base + b], n_idx[slot])
            pltpu.async_copy(iemb_hbm.at[n_idx[slot]], n_rows[slot],
                             sems[slot])

        def compute(b, slot):
            rows = n_rows[slot]
            u0 = u_rows[b, pl.ds(0, _L)]
            u1 = u_rows[b, pl.ds(_L, _L)]
            u2 = u_rows[b, pl.ds(2 * _L, _L)]
            u3 = u_rows[b, pl.ds(3 * _L, _L)]

            def dot_to(buf, r, j):
                t = (u0 * rows[j, pl.ds(0, _L)]
                     + u1 * rows[j, pl.ds(_L, _L)]
                     + u2 * rows[j, pl.ds(2 * _L, _L)]
                     + u3 * rows[j, pl.ds(3 * _L, _L)])
                rot_store(buf, r, t)

            def gbody(g, sacc):
                for r in range(_L):
                    dot_to(tbuf, r, g * _L + r)
                return sacc + jnp.exp(treduce(tbuf) * _INV_T)

            sacc = lax.fori_loop(0, n_full, gbody,
                                 jnp.zeros((_L,), jnp.float32), unroll=2)
            if n_tail:
                for r in range(n_tail):
                    dot_to(tbuf2, r, n_full * _L + r)
                sacc = sacc + jnp.exp(treduce(tbuf2) * _INV_T)

            tp = (u0 * p_rows[b, pl.ds(0, _L)]
                  + u1 * p_rows[b, pl.ds(_L, _L)]
                  + u2 * p_rows[b, pl.ds(2 * _L, _L)]
                  + u3 * p_rows[b, pl.ds(3 * _L, _L)])
            bm = jnp.bitwise_and(b, _L - 1)
            rbm = jnp.bitwise_and(lane + bm, _L - 1)
            sbuf[pl.ds(bm * _L, _L)] = sacc.at[rbm].get(mode="promise_in_bounds")
            pbuf[pl.ds(bm * _L, _L)] = tp.at[rbm].get(mode="promise_in_bounds")

            @pl.when(bm == _L - 1)
            def _fold():
                svec = treduce(sbuf)
                pe = jnp.exp(treduce(pbuf) * _INV_T)
                res[pl.ds(b - (_L - 1), _L)] = pe / (pe + svec)

        fetch(0, 0)

        def outer(i, _):
            for k in range(2):
                b = i * 2 + k
                slot = k

                @pl.when(b + 1 < bw)
                def _prefetch():
                    fetch(b + 1, 1 - slot)

                pltpu.make_async_copy(
                    iemb_hbm.at[n_idx[slot]], n_rows[slot],
                    sems[slot]).wait()
                compute(b, slot)
            return _

        lax.fori_loop(0, bw // 2, outer, 0, unroll=False)
        pltpu.sync_copy(res, out_hbm.at[pl.ds(base, bw)])

    return pl.kernel(
        body,
        out_type=jax.ShapeDtypeStruct((B,), jnp.float32),
        mesh=mesh,
        compiler_params=pltpu.CompilerParams(
            needs_layout_passes=False, use_tc_tiling_on_sc=False),
        scratch_types=[
            pltpu.VMEM((bw,), jnp.int32),
            pltpu.VMEM((bw,), jnp.int32),
            pltpu.VMEM((bw, D), jnp.float32),
            pltpu.VMEM((bw, D), jnp.float32),
            pltpu.VMEM((N,), jnp.int32),
            pltpu.VMEM((N,), jnp.int32),
            pltpu.VMEM((N, D), jnp.float32),
            pltpu.VMEM((N, D), jnp.float32),
            pltpu.VMEM((_L * _L,), jnp.float32),
            pltpu.VMEM((_L * _L,), jnp.float32),
            pltpu.VMEM((_L * _L,), jnp.float32),
            pltpu.VMEM((_L * _L,), jnp.float32),
            pltpu.VMEM((bw,), jnp.float32),
            pltpu.SemaphoreType.DMA,
            pltpu.SemaphoreType.DMA,
        ],
    )


def _loss_body(r_ref, o_ref):
    o_ref[0, 0] = -jnp.mean(jnp.log(r_ref[...]))


@functools.lru_cache(maxsize=None)
def _make_tc_loss(rows, cols):
    return pl.pallas_call(
        _loss_body,
        out_shape=jax.ShapeDtypeStruct((1, 1), jnp.float32),
        out_specs=pl.BlockSpec(memory_space=pltpu.SMEM),
    )


def kernel(users, positives, negatives, user_emb, item_emb, epoch):
    B = users.shape[0]
    N = negatives.shape[1]
    V, D = item_emb.shape
    info = plsc.get_sparse_core_info()
    NC, NS = info.num_cores, info.num_subcores
    tail = V % 256
    item_lin, user_lin = _make_fmt(V, D, NC, NS)(
        jnp.transpose(item_emb), jnp.transpose(user_emb),
        item_emb[V - tail:, :].reshape(-1), user_emb[V - tail:, :].reshape(-1))
    sc = _make_sc_scores(B, N, D, NC, NS)
    r = sc(users.astype(jnp.int32),
           positives.reshape(-1).astype(jnp.int32),
           negatives.astype(jnp.int32),
           user_lin.reshape(V, D), item_lin.reshape(V, D))
    loss = _make_tc_loss(B // 128, 128)(r.reshape(B // 128, 128))
    return loss[0, 0]
